# baseline (speedup 1.0000x reference)
import functools

import jax
import jax.numpy as jnp
from jax.experimental import pallas as pl
from jax.experimental.pallas import tpu as pltpu

_EMB_DIM = 384
_HEADS = 4
_LAMBDA = 5.0
_THRESH = 0.1
_SLOPE = 0.2
_LOG2E = 1.4426950408889634

_BLK = 256
_HALO = 16
_EXT = _BLK + 2 * _HALO


def _gat_band_kernel(emb_ref, w_ref, asrc_ref, adst_ref, bias_ref, out_ref,
                     adst_scr):
    i = pl.program_id(0)
    n = emb_ref.shape[0]
    c = jnp.minimum(jnp.maximum(i * _BLK - _HALO, 0), n - _EXT)
    c = pl.multiple_of(c, 8)
    o = i * _BLK - c

    emb_ext = emb_ref[pl.ds(c, _EXT), :]
    norms2_e = jnp.sum(emb_ext * emb_ext, axis=1, keepdims=True)
    en_ext = emb_ext * jax.lax.rsqrt(jnp.maximum(norms2_e, 1e-24))
    emb_blk = emb_ref[pl.ds(i * _BLK, _BLK), :]
    norms2_b = jnp.sum(emb_blk * emb_blk, axis=1, keepdims=True)
    en_blk = emb_blk * jax.lax.rsqrt(jnp.maximum(norms2_b, 1e-24))

    sim = jax.lax.dot_general(
        en_blk, en_ext, (((1,), (1,)), ((), ())),
        preferred_element_type=jnp.float32)

    rows = jax.lax.broadcasted_iota(jnp.int32, (_BLK, _EXT), 0)
    cols = jax.lax.broadcasted_iota(jnp.int32, (_BLK, _EXT), 1)
    dist = jnp.abs(rows + o - cols).astype(jnp.float32)
    mask = sim * jnp.exp2(dist * (-_LOG2E / _LAMBDA)) > _THRESH

    x_ext = jax.lax.dot_general(
        emb_ext, w_ref[...], (((1,), (0,)), ((), ())),
        preferred_element_type=jnp.float32)

    acc = jnp.zeros((_BLK, _EMB_DIM), dtype=jnp.float32)
    for h in range(_HEADS):
        xh = x_ext[:, h * _EMB_DIM:(h + 1) * _EMB_DIM]
        a_src = jax.lax.dot_general(
            asrc_ref[h:h + 1, :], xh, (((1,), (1,)), ((), ())),
            preferred_element_type=jnp.float32)
        a_dst_ext = jax.lax.dot_general(
            xh, adst_ref[h:h + 1, :], (((1,), (1,)), ((), ())),
            preferred_element_type=jnp.float32)
        adst_scr[:, h:h + 1] = a_dst_ext
        a_dst = adst_scr[pl.ds(o, _BLK), h:h + 1]
        logits = a_dst * _LOG2E + a_src * _LOG2E
        logits = jnp.maximum(logits, _SLOPE * logits)
        p = jnp.where(mask, jnp.exp2(logits), 0.0)
        denom = jnp.sum(p, axis=1, keepdims=True)
        y = jax.lax.dot_general(
            p, xh, (((1,), (0,)), ((), ())),
            preferred_element_type=jnp.float32)
        acc = acc + y * ((1.0 / _HEADS) / denom)

    out_ref[...] = acc + bias_ref[...][None, :]


@functools.partial(jax.jit, static_argnames=())
def kernel(embeddings, span_positions, W, att_src, att_dst, bias):
    del span_positions
    n, d = embeddings.shape
    grid = (n // _BLK,)
    out = pl.pallas_call(
        _gat_band_kernel,
        grid=grid,
        in_specs=[
            pl.BlockSpec((n, d), lambda i: (0, 0)),
            pl.BlockSpec(W.shape, lambda i: (0, 0)),
            pl.BlockSpec(att_src.shape, lambda i: (0, 0)),
            pl.BlockSpec(att_dst.shape, lambda i: (0, 0)),
            pl.BlockSpec(bias.shape, lambda i: (0,)),
        ],
        out_specs=pl.BlockSpec((_BLK, d), lambda i: (i, 0)),
        out_shape=jax.ShapeDtypeStruct((n, d), jnp.float32),
        scratch_shapes=[
            pltpu.VMEM((_EXT, _HEADS), jnp.float32),
        ],
    )(embeddings, W, att_src, att_dst, bias)
    return out


# speedup vs baseline: 1.0169x; 1.0169x over previous
import functools

import jax
import jax.numpy as jnp
from jax.experimental import pallas as pl
from jax.experimental.pallas import tpu as pltpu

_EMB_DIM = 384
_HEADS = 4
_LAMBDA = 5.0
_THRESH = 0.1
_SLOPE = 0.2

_LOG2E = 1.4426950408889634

_BLK = 256
_HALO = 16
_EXT = _BLK + 2 * _HALO


def _gat_band_kernel(emb_ref, w_ref, asrc_ref, adst_ref, bias_ref, out_ref,
                     pad_scr):
    i = pl.program_id(0)
    n = emb_ref.shape[0]

    @pl.when(i == 0)
    def _stage_padded():
        pad_scr[0:_HALO, :] = jnp.zeros((_HALO, _EMB_DIM), jnp.float32)
        pad_scr[pl.ds(_HALO, n), :] = emb_ref[...]
        pad_scr[pl.ds(n + _HALO, _HALO), :] = jnp.zeros(
            (_HALO, _EMB_DIM), jnp.float32)

    emb_ext = pad_scr[pl.ds(i * _BLK, _EXT), :]
    norms2 = jnp.sum(emb_ext * emb_ext, axis=1, keepdims=True)
    en_ext = emb_ext * jax.lax.rsqrt(jnp.maximum(norms2, 1e-24))
    en_blk = en_ext[_HALO:_HALO + _BLK, :]

    sim = jax.lax.dot_general(
        en_blk, en_ext, (((1,), (1,)), ((), ())),
        preferred_element_type=jnp.float32)

    rows = jax.lax.broadcasted_iota(jnp.int32, (_BLK, _EXT), 0)
    cols = jax.lax.broadcasted_iota(jnp.int32, (_BLK, _EXT), 1)
    dist = jnp.abs(rows + _HALO - cols).astype(jnp.float32)
    mask = sim * jnp.exp2(dist * (-_LOG2E / _LAMBDA)) > _THRESH

    x_ext = jax.lax.dot_general(
        emb_ext, w_ref[...], (((1,), (0,)), ((), ())),
        preferred_element_type=jnp.float32)

    acc = jnp.zeros((_BLK, _EMB_DIM), dtype=jnp.float32)
    for h in range(_HEADS):
        xh = x_ext[:, h * _EMB_DIM:(h + 1) * _EMB_DIM]
        xh_blk = xh[_HALO:_HALO + _BLK, :]
        a_src = jax.lax.dot_general(
            asrc_ref[h:h + 1, :], xh, (((1,), (1,)), ((), ())),
            preferred_element_type=jnp.float32)
        a_dst = jax.lax.dot_general(
            xh_blk, adst_ref[h:h + 1, :], (((1,), (1,)), ((), ())),
            preferred_element_type=jnp.float32)
        logits = a_dst * _LOG2E + a_src * _LOG2E
        logits = jnp.maximum(logits, _SLOPE * logits)
        p = jnp.where(mask, jnp.exp2(logits), 0.0)
        denom = jnp.sum(p, axis=1, keepdims=True)
        y = jax.lax.dot_general(
            p, xh, (((1,), (0,)), ((), ())),
            preferred_element_type=jnp.float32)
        acc = acc + y * ((1.0 / _HEADS) / denom)

    out_ref[...] = acc + bias_ref[...][None, :]


@functools.partial(jax.jit, static_argnames=())
def kernel(embeddings, span_positions, W, att_src, att_dst, bias):
    del span_positions
    n, d = embeddings.shape
    grid = (n // _BLK,)
    out = pl.pallas_call(
        _gat_band_kernel,
        grid=grid,
        in_specs=[
            pl.BlockSpec((n, d), lambda i: (0, 0)),
            pl.BlockSpec(W.shape, lambda i: (0, 0)),
            pl.BlockSpec(att_src.shape, lambda i: (0, 0)),
            pl.BlockSpec(att_dst.shape, lambda i: (0, 0)),
            pl.BlockSpec(bias.shape, lambda i: (0,)),
        ],
        out_specs=pl.BlockSpec((_BLK, d), lambda i: (i, 0)),
        out_shape=jax.ShapeDtypeStruct((n, d), jnp.float32),
        scratch_shapes=[
            pltpu.VMEM((n + 2 * _HALO, d), jnp.float32),
        ],
    )(embeddings, W, att_src, att_dst, bias)
    return out
